# baseline (speedup 1.0000x reference)
import jax
import jax.numpy as jnp
from jax import lax
from jax.experimental import pallas as pl
from jax.experimental.pallas import tpu as pltpu
from jax.experimental.pallas import tpu_sc as plsc

_NC, _NS = 2, 16
_NW = _NC * _NS
_TOTAL = 16 * 3 * 512 * 512

_TEC_PER = 229_376
_TEC_CH = 28_672
_TEC_NCH = _TEC_PER // _TEC_CH
_TEC_NBUF = 2
_TEC_TOTAL = _TEC_PER * _NW

_SCS_PER = 2_621_440
_SCS_CH = 262_144
_SCS_NCH = _SCS_PER // _SCS_CH
_SCS_NBUF = 3


def _tec_fn(x_hbm, out_hbm, tbuf, tis, tos, b0, b1, b2, sis, sos):
    del b0, b1, b2, sis, sos
    wid = lax.axis_index("s") * _NC + lax.axis_index("c")
    base = wid * _TEC_PER

    def cin(i, b):
        return pltpu.async_copy(
            x_hbm.at[pl.ds(base + i * _TEC_CH, _TEC_CH)], tbuf.at[b],
            tis.at[b])

    def cout(i, b):
        return pltpu.async_copy(
            tbuf.at[b], out_hbm.at[pl.ds(base + i * _TEC_CH, _TEC_CH)],
            tos.at[b])

    ins, outs = {}, {}
    for i in range(_TEC_NBUF):
        ins[i] = cin(i, i)
    for i in range(_TEC_NCH):
        b = i % _TEC_NBUF
        ins[i].wait()
        outs[i] = cout(i, b)
        j = i + _TEC_NBUF
        if j < _TEC_NCH:
            outs[i].wait()
            ins[j] = cin(j, b)
    for i in range(max(_TEC_NCH - _TEC_NBUF, 0), _TEC_NCH):
        outs[i].wait()


def _scs_fn(x_hbm, out_hbm, tbuf, tis, tos, b0, b1, b2, sis, sos):
    del tbuf, tis, tos
    cid = lax.axis_index("c")
    base = _TEC_TOTAL + cid * _SCS_PER
    bufs = [b0, b1, b2]

    def cin(i):
        return pltpu.async_copy(
            x_hbm.at[pl.ds(base + i * _SCS_CH, _SCS_CH)],
            bufs[i % _SCS_NBUF], sis.at[i % _SCS_NBUF])

    def cout(i):
        return pltpu.async_copy(
            bufs[i % _SCS_NBUF],
            out_hbm.at[pl.ds(base + i * _SCS_CH, _SCS_CH)],
            sos.at[i % _SCS_NBUF])

    ins, outs = {}, {}
    for i in range(_SCS_NBUF):
        ins[i] = cin(i)
    for i in range(_SCS_NCH):
        ins[i].wait()
        outs[i] = cout(i)
        j = i + _SCS_NBUF
        if j < _SCS_NCH:
            outs[i].wait()
            ins[j] = cin(j)
    for i in range(max(_SCS_NCH - _SCS_NBUF, 0), _SCS_NCH):
        outs[i].wait()


_VMESH = plsc.VectorSubcoreMesh(core_axis_name="c", subcore_axis_name="s")
_SMESH = plsc.ScalarSubcoreMesh(axis_name="c", num_cores=_NC)
_DMA_DT = pltpu.SemaphoreType.DMA.dtype

_sc_copy = pl.kernel(
    body=[_tec_fn, _scs_fn],
    out_type=jax.ShapeDtypeStruct((_TOTAL,), jnp.float32),
    mesh=[_VMESH, _SMESH],
    scratch_types=[
        (pltpu.MemorySpace.VMEM @ _VMESH)((_TEC_NBUF, _TEC_CH), jnp.float32),
        (pltpu.MemorySpace.SEMAPHORE @ _VMESH)((_TEC_NBUF,), _DMA_DT),
        (pltpu.MemorySpace.SEMAPHORE @ _VMESH)((_TEC_NBUF,), _DMA_DT),
        pltpu.MemorySpace.VMEM_SHARED((_SCS_CH,), jnp.float32),
        pltpu.MemorySpace.VMEM_SHARED((_SCS_CH,), jnp.float32),
        pltpu.MemorySpace.VMEM_SHARED((_SCS_CH,), jnp.float32),
        (pltpu.MemorySpace.SEMAPHORE @ _SMESH)((_SCS_NBUF,), _DMA_DT),
        (pltpu.MemorySpace.SEMAPHORE @ _SMESH)((_SCS_NBUF,), _DMA_DT),
    ],
)


def kernel(x):
    out = _sc_copy(x.reshape(-1))
    return out.reshape(x.shape)
